# baseline (speedup 1.0000x reference)
import functools

import jax
import jax.numpy as jnp
from jax import lax
from jax.experimental import pallas as pl
from jax.experimental.pallas import tpu as pltpu
from jax.experimental.pallas import tpu_sc as plsc

_EPS = 1e-5

_NUM_HW = 100
_EMBED_DIM = 64
_BATCH = 16384

_info = plsc.get_sparse_core_info()
_NC, _NS = _info.num_cores, _info.num_subcores
_NW = _NC * _NS
_B_PER_W = _BATCH // _NW
_LANES = 16
_NJ = _EMBED_DIM // _LANES

_mesh = plsc.VectorSubcoreMesh(core_axis_name="c", subcore_axis_name="s")


@functools.partial(
    pl.kernel,
    mesh=_mesh,
    out_type=jax.ShapeDtypeStruct((_BATCH, _EMBED_DIM), jnp.float32),
    scratch_types=[
        pltpu.VMEM((_NUM_HW, _EMBED_DIM), jnp.float32),
        pltpu.VMEM((_EMBED_DIM,), jnp.float32),
        pltpu.VMEM((_EMBED_DIM,), jnp.float32),
        pltpu.VMEM((_B_PER_W,), jnp.int32),
        pltpu.VMEM((_B_PER_W, _EMBED_DIM), jnp.float32),
        pltpu.SemaphoreType.DMA,
    ],
    compiler_params=pltpu.CompilerParams(
        use_tc_tiling_on_sc=False, needs_layout_passes=False),
)
def _sc_fused(idx_hbm, table_hbm, gamma_hbm, beta_hbm, out_hbm,
              tbl_v, g_v, b_v, idx_v, rows_v, sem):
    wid = lax.axis_index("s") * _NC + lax.axis_index("c")
    base = wid * _B_PER_W
    cp_idx = pltpu.async_copy(idx_hbm.at[pl.ds(base, _B_PER_W)], idx_v, sem)
    pltpu.sync_copy(table_hbm, tbl_v)
    pltpu.sync_copy(gamma_hbm, g_v)
    pltpu.sync_copy(beta_hbm, b_v)

    g = [g_v[pl.ds(_LANES * j, _LANES)] for j in range(_NJ)]
    b = [b_v[pl.ds(_LANES * j, _LANES)] for j in range(_NJ)]

    inv_d = jnp.float32(1.0 / _EMBED_DIM)

    def norm_row(r, carry):
        x = [tbl_v[r, pl.ds(_LANES * j, _LANES)] for j in range(_NJ)]
        mean = jnp.sum(x[0] + x[1] + x[2] + x[3]) * inv_d
        c = [xj - mean for xj in x]
        var = jnp.sum(c[0] * c[0] + c[1] * c[1] + c[2] * c[2] + c[3] * c[3]) * inv_d
        v = jax.lax.broadcast(var + jnp.float32(_EPS), (_LANES,))
        i = plsc.bitcast(v, jnp.int32)
        i = jnp.int32(0x5F3759DF) - (i >> 1)
        y = plsc.bitcast(i, jnp.float32)
        for _ in range(3):
            y = y * (jnp.float32(1.5) - jnp.float32(0.5) * v * y * y)
        for j in range(_NJ):
            tbl_v[r, pl.ds(_LANES * j, _LANES)] = c[j] * y * g[j] + b[j]
        return carry

    lax.fori_loop(0, _NUM_HW, norm_row, 0)

    cp_idx.wait()

    def expand(gidx, carry):
        base_b = gidx * _LANES
        v16 = idx_v[pl.ds(base_b, _LANES)]
        for k in range(_LANES):
            r = v16[k]
            bi = base_b + k
            for j in range(_NJ):
                rows_v[bi, pl.ds(_LANES * j, _LANES)] = (
                    tbl_v[r, pl.ds(_LANES * j, _LANES)])
        return carry

    lax.fori_loop(0, _B_PER_W // _LANES, expand, 0)

    pltpu.sync_copy(rows_v, out_hbm.at[pl.ds(base, _B_PER_W)])


def kernel(hw_indices, table, gamma, beta):
    return _sc_fused(hw_indices.astype(jnp.int32), table, gamma, beta)


# speedup vs baseline: 1.0997x; 1.0997x over previous
import functools

import jax
import jax.numpy as jnp
from jax import lax
from jax.experimental import pallas as pl
from jax.experimental.pallas import tpu as pltpu
from jax.experimental.pallas import tpu_sc as plsc

_EPS = 1e-5

_NUM_HW = 100
_EMBED_DIM = 64
_BATCH = 16384

_info = plsc.get_sparse_core_info()
_NC, _NS = _info.num_cores, _info.num_subcores
_NW = _NC * _NS
_B_PER_W = _BATCH // _NW
_LANES = 16
_NJ = _EMBED_DIM // _LANES

_mesh = plsc.VectorSubcoreMesh(core_axis_name="c", subcore_axis_name="s")


@functools.partial(
    pl.kernel,
    mesh=_mesh,
    out_type=jax.ShapeDtypeStruct((_BATCH, _EMBED_DIM), jnp.float32),
    scratch_types=[
        pltpu.VMEM((_NUM_HW, _EMBED_DIM), jnp.float32),
        pltpu.VMEM((_EMBED_DIM,), jnp.float32),
        pltpu.VMEM((_EMBED_DIM,), jnp.float32),
        pltpu.VMEM((_B_PER_W,), jnp.int32),
        pltpu.VMEM((_B_PER_W, _EMBED_DIM), jnp.float32),
        pltpu.SemaphoreType.DMA,
    ],
    compiler_params=pltpu.CompilerParams(
        use_tc_tiling_on_sc=True, needs_layout_passes=False),
)
def _sc_fused(idx_hbm, table_hbm, gamma_hbm, beta_hbm, out_hbm,
              tbl_v, g_v, b_v, idx_v, rows_v, sem):
    wid = lax.axis_index("s") * _NC + lax.axis_index("c")
    base = wid * _B_PER_W
    cp_idx = pltpu.async_copy(idx_hbm.at[pl.ds(base, _B_PER_W)], idx_v, sem)
    pltpu.sync_copy(table_hbm, tbl_v)
    pltpu.sync_copy(gamma_hbm, g_v)
    pltpu.sync_copy(beta_hbm, b_v)

    g = [g_v[pl.ds(_LANES * j, _LANES)] for j in range(_NJ)]
    b = [b_v[pl.ds(_LANES * j, _LANES)] for j in range(_NJ)]

    inv_d = jnp.float32(1.0 / _EMBED_DIM)

    def norm_row(r, carry):
        x = [tbl_v[r, pl.ds(_LANES * j, _LANES)] for j in range(_NJ)]
        mean = jnp.sum(x[0] + x[1] + x[2] + x[3]) * inv_d
        c = [xj - mean for xj in x]
        var = jnp.sum(c[0] * c[0] + c[1] * c[1] + c[2] * c[2] + c[3] * c[3]) * inv_d
        v = jax.lax.broadcast(var + jnp.float32(_EPS), (_LANES,))
        i = plsc.bitcast(v, jnp.int32)
        i = jnp.int32(0x5F3759DF) - (i >> 1)
        y = plsc.bitcast(i, jnp.float32)
        for _ in range(3):
            y = y * (jnp.float32(1.5) - jnp.float32(0.5) * v * y * y)
        for j in range(_NJ):
            tbl_v[r, pl.ds(_LANES * j, _LANES)] = c[j] * y * g[j] + b[j]
        return carry

    lax.fori_loop(0, _NUM_HW, norm_row, 0)

    cp_idx.wait()

    def expand(gidx, carry):
        base_b = gidx * _LANES
        v16 = idx_v[pl.ds(base_b, _LANES)]
        for k in range(_LANES):
            r = v16[k]
            bi = base_b + k
            for j in range(_NJ):
                rows_v[bi, pl.ds(_LANES * j, _LANES)] = (
                    tbl_v[r, pl.ds(_LANES * j, _LANES)])
        return carry

    lax.fori_loop(0, _B_PER_W // _LANES, expand, 0)

    pltpu.sync_copy(rows_v, out_hbm.at[pl.ds(base, _B_PER_W)])


def kernel(hw_indices, table, gamma, beta):
    return _sc_fused(hw_indices.astype(jnp.int32), table, gamma, beta)


# speedup vs baseline: 1.4240x; 1.2949x over previous
import functools

import jax
import jax.numpy as jnp
from jax import lax
from jax.experimental import pallas as pl
from jax.experimental.pallas import tpu as pltpu
from jax.experimental.pallas import tpu_sc as plsc

_EPS = 1e-5

_NUM_HW = 100
_NUM_HW_PAD = 112
_EMBED_DIM = 64
_BATCH = 16384

_info = plsc.get_sparse_core_info()
_NC, _NS = _info.num_cores, _info.num_subcores
_NW = _NC * _NS
_B_PER_W = _BATCH // _NW
_LANES = 16
_NG = _NUM_HW_PAD // _LANES
_NBG = _B_PER_W // _LANES

_mesh = plsc.VectorSubcoreMesh(core_axis_name="c", subcore_axis_name="s")


@functools.partial(
    pl.kernel,
    mesh=_mesh,
    out_type=jax.ShapeDtypeStruct((_EMBED_DIM, _BATCH), jnp.float32),
    scratch_types=[
        pltpu.VMEM((_EMBED_DIM, _NUM_HW_PAD), jnp.float32),
        pltpu.VMEM((_EMBED_DIM,), jnp.float32),
        pltpu.VMEM((_EMBED_DIM,), jnp.float32),
        pltpu.VMEM((_B_PER_W,), jnp.int32),
        pltpu.VMEM((_EMBED_DIM, _B_PER_W), jnp.float32),
        pltpu.SemaphoreType.DMA,
    ],
    compiler_params=pltpu.CompilerParams(
        use_tc_tiling_on_sc=True, needs_layout_passes=False),
)
def _sc_fused(idx_hbm, tablet_hbm, gamma_hbm, beta_hbm, out_hbm,
              tbl_t, g_v, b_v, idx_v, rows_v, sem):
    wid = lax.axis_index("s") * _NC + lax.axis_index("c")
    base = wid * _B_PER_W
    cp_idx = pltpu.async_copy(idx_hbm.at[pl.ds(base, _B_PER_W)], idx_v, sem)
    pltpu.sync_copy(tablet_hbm, tbl_t)
    pltpu.sync_copy(gamma_hbm, g_v)
    pltpu.sync_copy(beta_hbm, b_v)

    half = jnp.float32(0.5)
    threehalf = jnp.float32(1.5)
    inv_d = jnp.float32(1.0 / _EMBED_DIM)

    sums = []
    sqs = []
    for gi in range(_NG):
        s = tbl_t[0, pl.ds(_LANES * gi, _LANES)]
        q = s * s
        sums.append(s)
        sqs.append(q)
    for d in range(1, _EMBED_DIM):
        for gi in range(_NG):
            v = tbl_t[d, pl.ds(_LANES * gi, _LANES)]
            sums[gi] = sums[gi] + v
            sqs[gi] = sqs[gi] + v * v
    means = [s * inv_d for s in sums]
    rstds = []
    for gi in range(_NG):
        var = sqs[gi] * inv_d - means[gi] * means[gi]
        v = var + jnp.float32(_EPS)
        i = plsc.bitcast(v, jnp.int32)
        i = jnp.int32(0x5F3759DF) - (i >> 1)
        y = plsc.bitcast(i, jnp.float32)
        for _ in range(3):
            y = y * (threehalf - half * v * y * y)
        rstds.append(y)

    for dj in range(_EMBED_DIM // _LANES):
        g16 = g_v[pl.ds(_LANES * dj, _LANES)]
        b16 = b_v[pl.ds(_LANES * dj, _LANES)]
        for k in range(_LANES):
            d = _LANES * dj + k
            gd = g16[k]
            bd = b16[k]
            for gi in range(_NG):
                x = tbl_t[d, pl.ds(_LANES * gi, _LANES)]
                tbl_t[d, pl.ds(_LANES * gi, _LANES)] = (
                    (x - means[gi]) * rstds[gi] * gd + bd)

    cp_idx.wait()

    def expand(bg, carry):
        bo = bg * _LANES
        idx16 = idx_v[pl.ds(bo, _LANES)]
        for d in range(_EMBED_DIM):
            vals = plsc.load_gather(
                tbl_t, [jnp.full((_LANES,), d, jnp.int32), idx16])
            rows_v[d, pl.ds(bo, _LANES)] = vals
        return carry

    lax.fori_loop(0, _NBG, expand, 0)

    pltpu.sync_copy(rows_v, out_hbm.at[:, pl.ds(base, _B_PER_W)])


def kernel(hw_indices, table, gamma, beta):
    tablet = jnp.pad(table.T, ((0, 0), (0, _NUM_HW_PAD - _NUM_HW)))
    out_t = _sc_fused(hw_indices.astype(jnp.int32), tablet, gamma, beta)
    return out_t.T


# speedup vs baseline: 1.4288x; 1.0033x over previous
import functools

import jax
import jax.numpy as jnp
from jax import lax
from jax.experimental import pallas as pl
from jax.experimental.pallas import tpu as pltpu
from jax.experimental.pallas import tpu_sc as plsc

_EPS = 1e-5

_NUM_HW = 100
_NUM_HW_PAD = 112
_EMBED_DIM = 64
_BATCH = 16384

_info = plsc.get_sparse_core_info()
_NC, _NS = _info.num_cores, _info.num_subcores
_NW = _NC * _NS
_B_PER_W = _BATCH // _NW
_LANES = 16
_NG = _NUM_HW_PAD // _LANES
_NBG = _B_PER_W // _LANES

_mesh = plsc.VectorSubcoreMesh(core_axis_name="c", subcore_axis_name="s")


@functools.partial(
    pl.kernel,
    mesh=_mesh,
    out_type=jax.ShapeDtypeStruct((_EMBED_DIM, _BATCH), jnp.float32),
    scratch_types=[
        pltpu.VMEM((_EMBED_DIM, _NUM_HW_PAD), jnp.float32),
        pltpu.VMEM((_EMBED_DIM,), jnp.float32),
        pltpu.VMEM((_EMBED_DIM,), jnp.float32),
        pltpu.VMEM((_B_PER_W,), jnp.int32),
        pltpu.VMEM((_EMBED_DIM, _B_PER_W), jnp.float32),
        pltpu.SemaphoreType.DMA,
    ],
    compiler_params=pltpu.CompilerParams(
        use_tc_tiling_on_sc=True, needs_layout_passes=False),
)
def _sc_fused(idx_hbm, tablet_hbm, gamma_hbm, beta_hbm, out_hbm,
              tbl_t, g_v, b_v, idx_v, rows_v, sem):
    wid = lax.axis_index("s") * _NC + lax.axis_index("c")
    base = wid * _B_PER_W
    with jax.named_scope("stage"):
        cp_idx = pltpu.async_copy(idx_hbm.at[pl.ds(base, _B_PER_W)], idx_v, sem)
        pltpu.sync_copy(tablet_hbm, tbl_t)
        pltpu.sync_copy(gamma_hbm, g_v)
        pltpu.sync_copy(beta_hbm, b_v)

    ln_scope = jax.named_scope("normalize"); ln_scope.__enter__()
    half = jnp.float32(0.5)
    threehalf = jnp.float32(1.5)
    inv_d = jnp.float32(1.0 / _EMBED_DIM)

    sums = []
    sqs = []
    for gi in range(_NG):
        s = tbl_t[0, pl.ds(_LANES * gi, _LANES)]
        q = s * s
        sums.append(s)
        sqs.append(q)
    for d in range(1, _EMBED_DIM):
        for gi in range(_NG):
            v = tbl_t[d, pl.ds(_LANES * gi, _LANES)]
            sums[gi] = sums[gi] + v
            sqs[gi] = sqs[gi] + v * v
    means = [s * inv_d for s in sums]
    rstds = []
    for gi in range(_NG):
        var = sqs[gi] * inv_d - means[gi] * means[gi]
        v = var + jnp.float32(_EPS)
        i = plsc.bitcast(v, jnp.int32)
        i = jnp.int32(0x5F3759DF) - (i >> 1)
        y = plsc.bitcast(i, jnp.float32)
        for _ in range(3):
            y = y * (threehalf - half * v * y * y)
        rstds.append(y)

    for dj in range(_EMBED_DIM // _LANES):
        g16 = g_v[pl.ds(_LANES * dj, _LANES)]
        b16 = b_v[pl.ds(_LANES * dj, _LANES)]
        for k in range(_LANES):
            d = _LANES * dj + k
            gd = g16[k]
            bd = b16[k]
            for gi in range(_NG):
                x = tbl_t[d, pl.ds(_LANES * gi, _LANES)]
                tbl_t[d, pl.ds(_LANES * gi, _LANES)] = (
                    (x - means[gi]) * rstds[gi] * gd + bd)

    ln_scope.__exit__(None, None, None)

    cp_idx.wait()

    def expand(bg, carry):
        bo = bg * _LANES
        idx16 = idx_v[pl.ds(bo, _LANES)]
        for d in range(_EMBED_DIM):
            vals = plsc.load_gather(
                tbl_t, [jnp.full((_LANES,), d, jnp.int32), idx16])
            rows_v[d, pl.ds(bo, _LANES)] = vals
        return carry

    with jax.named_scope("expand"):
        lax.fori_loop(0, _NBG, expand, 0)

    with jax.named_scope("flush"):
        pltpu.sync_copy(rows_v, out_hbm.at[:, pl.ds(base, _B_PER_W)])


def kernel(hw_indices, table, gamma, beta):
    tablet = jnp.pad(table.T, ((0, 0), (0, _NUM_HW_PAD - _NUM_HW)))
    out_t = _sc_fused(hw_indices.astype(jnp.int32), tablet, gamma, beta)
    return out_t.T


# speedup vs baseline: 1.4690x; 1.0282x over previous
import functools

import jax
import jax.numpy as jnp
from jax import lax
from jax.experimental import pallas as pl
from jax.experimental.pallas import tpu as pltpu
from jax.experimental.pallas import tpu_sc as plsc

_EPS = 1e-5

_NUM_HW = 100
_NUM_HW_PAD = 112
_EMBED_DIM = 64
_BATCH = 16384

_info = plsc.get_sparse_core_info()
_NC, _NS = _info.num_cores, _info.num_subcores
_NW = _NC * _NS
_B_PER_W = _BATCH // _NW
_LANES = 16
_NG = _NUM_HW_PAD // _LANES
_NCHUNK = 4
_B_CHUNK = _B_PER_W // _NCHUNK
_G_CHUNK = _B_CHUNK // _LANES
_TBL_WORDS = _EMBED_DIM * _NUM_HW_PAD

_mesh = plsc.VectorSubcoreMesh(core_axis_name="c", subcore_axis_name="s")


@functools.partial(
    pl.kernel,
    mesh=_mesh,
    out_type=jax.ShapeDtypeStruct((_EMBED_DIM, _BATCH), jnp.float32),
    scratch_types=[
        pltpu.VMEM_SHARED((_TBL_WORDS,), jnp.float32),
        pltpu.VMEM((_TBL_WORDS,), jnp.float32),
        pltpu.VMEM((_EMBED_DIM,), jnp.float32),
        pltpu.VMEM((_EMBED_DIM,), jnp.float32),
        pltpu.VMEM((_B_PER_W,), jnp.int32),
        pltpu.VMEM((_EMBED_DIM, _B_PER_W), jnp.float32),
        pltpu.SemaphoreType.DMA,
        pltpu.SemaphoreType.DMA,
    ],
    compiler_params=pltpu.CompilerParams(
        use_tc_tiling_on_sc=True, needs_layout_passes=False),
)
def _sc_fused(idx_hbm, tablet_hbm, gamma_hbm, beta_hbm, out_hbm,
              tbl_s, tbl_f, g_v, b_v, idx_v, rows_v, sem, osem):
    sid = lax.axis_index("s")
    wid = sid * _NC + lax.axis_index("c")
    base = wid * _B_PER_W

    with jax.named_scope("stage"):
        cp_idx = pltpu.async_copy(idx_hbm.at[pl.ds(base, _B_PER_W)], idx_v, sem)

        @pl.when(sid == 0)
        def _():
            pltpu.sync_copy(tablet_hbm, tbl_s)

        pltpu.sync_copy(gamma_hbm, g_v)
        pltpu.sync_copy(beta_hbm, b_v)
        plsc.subcore_barrier()
        pltpu.sync_copy(tbl_s, tbl_f)

    ln_scope = jax.named_scope("normalize")
    ln_scope.__enter__()
    half = jnp.float32(0.5)
    threehalf = jnp.float32(1.5)
    inv_d = jnp.float32(1.0 / _EMBED_DIM)

    def cell(d, gi):
        return pl.ds(_NUM_HW_PAD * d + _LANES * gi, _LANES)

    sums = []
    sqs = []
    for gi in range(_NG):
        s = tbl_f[cell(0, gi)]
        sums.append(s)
        sqs.append(s * s)
    for d in range(1, _EMBED_DIM):
        for gi in range(_NG):
            v = tbl_f[cell(d, gi)]
            sums[gi] = sums[gi] + v
            sqs[gi] = sqs[gi] + v * v
    means = [s * inv_d for s in sums]
    rstds = []
    for gi in range(_NG):
        var = sqs[gi] * inv_d - means[gi] * means[gi]
        v = var + jnp.float32(_EPS)
        i = plsc.bitcast(v, jnp.int32)
        i = jnp.int32(0x5F3759DF) - (i >> 1)
        y = plsc.bitcast(i, jnp.float32)
        for _ in range(3):
            y = y * (threehalf - half * v * y * y)
        rstds.append(y)

    for dj in range(_EMBED_DIM // _LANES):
        g16 = g_v[pl.ds(_LANES * dj, _LANES)]
        b16 = b_v[pl.ds(_LANES * dj, _LANES)]
        for k in range(_LANES):
            d = _LANES * dj + k
            gd = g16[k]
            bd = b16[k]
            for gi in range(_NG):
                x = tbl_f[cell(d, gi)]
                tbl_f[cell(d, gi)] = (x - means[gi]) * rstds[gi] * gd + bd
    ln_scope.__exit__(None, None, None)

    cp_idx.wait()

    copies = []
    for c in range(_NCHUNK):
        with jax.named_scope("expand"):
            @plsc.parallel_loop(c * _G_CHUNK, (c + 1) * _G_CHUNK)
            def _(bg):
                bo = bg * _LANES
                idx16 = idx_v[pl.ds(bo, _LANES)]
                for d in range(_EMBED_DIM):
                    vals = plsc.load_gather(
                        tbl_f, [idx16 + jnp.int32(_NUM_HW_PAD * d)])
                    rows_v[d, pl.ds(bo, _LANES)] = vals

        with jax.named_scope("flush"):
            copies.append(pltpu.async_copy(
                rows_v.at[:, pl.ds(c * _B_CHUNK, _B_CHUNK)],
                out_hbm.at[:, pl.ds(base + c * _B_CHUNK, _B_CHUNK)],
                osem))
    with jax.named_scope("drain"):
        for cp in copies:
            cp.wait()


def kernel(hw_indices, table, gamma, beta):
    tablet = jnp.pad(table.T, ((0, 0), (0, _NUM_HW_PAD - _NUM_HW)))
    out_t = _sc_fused(
        hw_indices.astype(jnp.int32), tablet.reshape(_TBL_WORDS), gamma, beta)
    return out_t.T


# speedup vs baseline: 1.6166x; 1.1005x over previous
import functools

import jax
import jax.numpy as jnp
from jax import lax
from jax.experimental import pallas as pl
from jax.experimental.pallas import tpu as pltpu
from jax.experimental.pallas import tpu_sc as plsc

_EPS = 1e-5

_NUM_HW = 100
_EMBED_DIM = 64
_BATCH = 16384

_info = plsc.get_sparse_core_info()
_NC, _NS = _info.num_cores, _info.num_subcores
_NW = _NC * _NS
_B_PER_W = _BATCH // _NW
_LANES = 16
_NFULL = _NUM_HW // _LANES
_OVER = _NUM_HW - _LANES
_NG = _NFULL + 1
_NCHUNK = 4
_B_CHUNK = _B_PER_W // _NCHUNK
_G_CHUNK = _B_CHUNK // _LANES
_TBL_WORDS = _EMBED_DIM * _NUM_HW
_STAGE_W = _TBL_WORDS // _NS

_mesh = plsc.VectorSubcoreMesh(core_axis_name="c", subcore_axis_name="s")


def _group_off(gi):
    return _LANES * gi if gi < _NFULL else _OVER


@functools.partial(
    pl.kernel,
    mesh=_mesh,
    out_type=jax.ShapeDtypeStruct((_EMBED_DIM, _BATCH), jnp.float32),
    scratch_types=[
        pltpu.VMEM_SHARED((_TBL_WORDS,), jnp.float32),
        pltpu.VMEM((_TBL_WORDS,), jnp.float32),
        pltpu.VMEM((_EMBED_DIM,), jnp.float32),
        pltpu.VMEM((_EMBED_DIM,), jnp.float32),
        pltpu.VMEM((_B_PER_W,), jnp.int32),
        pltpu.VMEM((_EMBED_DIM, _B_PER_W), jnp.float32),
        pltpu.SemaphoreType.DMA,
        pltpu.SemaphoreType.DMA,
    ],
    compiler_params=pltpu.CompilerParams(
        use_tc_tiling_on_sc=True, needs_layout_passes=False),
)
def _sc_fused(idx_hbm, tablet_hbm, gamma_hbm, beta_hbm, out_hbm,
              tbl_s, tbl_f, g_v, b_v, idx_v, rows_v, sem, osem):
    sid = lax.axis_index("s")
    wid = sid * _NC + lax.axis_index("c")
    base = wid * _B_PER_W

    with jax.named_scope("stage"):
        cp_idx = pltpu.async_copy(idx_hbm.at[pl.ds(base, _B_PER_W)], idx_v, sem)
        soff = sid * _STAGE_W
        pltpu.sync_copy(tablet_hbm.at[pl.ds(soff, _STAGE_W)],
                        tbl_f.at[pl.ds(0, _STAGE_W)])
        pltpu.sync_copy(tbl_f.at[pl.ds(0, _STAGE_W)],
                        tbl_s.at[pl.ds(soff, _STAGE_W)])
        pltpu.sync_copy(gamma_hbm, g_v)
        pltpu.sync_copy(beta_hbm, b_v)
        plsc.subcore_barrier()
        pltpu.sync_copy(tbl_s, tbl_f)

    ln_scope = jax.named_scope("normalize")
    ln_scope.__enter__()
    half = jnp.float32(0.5)
    threehalf = jnp.float32(1.5)
    inv_d = jnp.float32(1.0 / _EMBED_DIM)
    zeros = jnp.zeros((_LANES,), jnp.float32)

    @plsc.parallel_loop(0, _EMBED_DIM, carry=tuple([zeros] * (2 * _NG)))
    def stats(d, acc):
        row = d * _NUM_HW
        out = []
        for gi in range(_NG):
            v = tbl_f[pl.ds(row + _group_off(gi), _LANES)]
            out.append(acc[gi] + v)
            out.append(acc[_NG + gi] + v * v)
        return tuple(out[0::2] + out[1::2])

    means = [stats[gi] * inv_d for gi in range(_NG)]
    rstds = []
    for gi in range(_NG):
        var = stats[_NG + gi] * inv_d - means[gi] * means[gi]
        v = var + jnp.float32(_EPS)
        i = plsc.bitcast(v, jnp.int32)
        i = jnp.int32(0x5F3759DF) - (i >> 1)
        y = plsc.bitcast(i, jnp.float32)
        for _ in range(3):
            y = y * (threehalf - half * v * y * y)
        rstds.append(y)

    tail_idx = lax.iota(jnp.int32, _LANES)
    tail_mask = tail_idx >= jnp.int32(_LANES - (_NUM_HW - _NFULL * _LANES))

    @plsc.parallel_loop(0, _EMBED_DIM)
    def _(d):
        row = d * _NUM_HW
        d16 = jnp.full((_LANES,), d, jnp.int32)
        gd = plsc.load_gather(g_v, [d16])
        bd = plsc.load_gather(b_v, [d16])
        for gi in range(_NFULL):
            off = row + _LANES * gi
            x = tbl_f[pl.ds(off, _LANES)]
            tbl_f[pl.ds(off, _LANES)] = (x - means[gi]) * rstds[gi] * gd + bd
        x = tbl_f[pl.ds(row + _OVER, _LANES)]
        y = (x - means[_NFULL]) * rstds[_NFULL] * gd + bd
        plsc.store_scatter(tbl_f, [tail_idx + (row + _OVER)], y, mask=tail_mask)

    ln_scope.__exit__(None, None, None)

    cp_idx.wait()

    copies = []
    for c in range(_NCHUNK):
        with jax.named_scope("expand"):
            @plsc.parallel_loop(c * _G_CHUNK, (c + 1) * _G_CHUNK)
            def _(bg):
                bo = bg * _LANES
                idx16 = idx_v[pl.ds(bo, _LANES)]
                for d in range(_EMBED_DIM):
                    vals = plsc.load_gather(
                        tbl_f, [idx16 + jnp.int32(_NUM_HW * d)])
                    rows_v[d, pl.ds(bo, _LANES)] = vals

        with jax.named_scope("flush"):
            copies.append(pltpu.async_copy(
                rows_v.at[:, pl.ds(c * _B_CHUNK, _B_CHUNK)],
                out_hbm.at[:, pl.ds(base + c * _B_CHUNK, _B_CHUNK)],
                osem))
    with jax.named_scope("drain"):
        for cp in copies:
            cp.wait()


def kernel(hw_indices, table, gamma, beta):
    out_t = _sc_fused(
        hw_indices.astype(jnp.int32),
        table.T.reshape(_TBL_WORDS), gamma, beta)
    return out_t.T


# speedup vs baseline: 1.6781x; 1.0380x over previous
import functools

import jax
import jax.numpy as jnp
from jax import lax
from jax.experimental import pallas as pl
from jax.experimental.pallas import tpu as pltpu
from jax.experimental.pallas import tpu_sc as plsc

_EPS = 1e-5

_NUM_HW = 100
_EMBED_DIM = 64
_BATCH = 16384

_info = plsc.get_sparse_core_info()
_NC, _NS = _info.num_cores, _info.num_subcores
_NW = _NC * _NS
_B_PER_W = _BATCH // _NW
_LANES = 16
_NFULL = _NUM_HW // _LANES
_OVER = _NUM_HW - _LANES
_NG = _NFULL + 1
_NCHUNK = 4
_B_CHUNK = _B_PER_W // _NCHUNK
_G_CHUNK = _B_CHUNK // _LANES

_mesh = plsc.VectorSubcoreMesh(core_axis_name="c", subcore_axis_name="s")


def _group_off(gi):
    return _LANES * gi if gi < _NFULL else _OVER


@functools.partial(
    pl.kernel,
    mesh=_mesh,
    out_type=jax.ShapeDtypeStruct((_EMBED_DIM, _BATCH), jnp.float32),
    scratch_types=[
        pltpu.VMEM((_EMBED_DIM, _NUM_HW), jnp.float32),
        pltpu.VMEM((_EMBED_DIM,), jnp.float32),
        pltpu.VMEM((_EMBED_DIM,), jnp.float32),
        pltpu.VMEM((_B_PER_W,), jnp.int32),
        pltpu.VMEM((_EMBED_DIM, _B_PER_W), jnp.float32),
        pltpu.SemaphoreType.DMA,
        pltpu.SemaphoreType.DMA,
        pltpu.SemaphoreType.DMA,
    ],
    compiler_params=pltpu.CompilerParams(
        use_tc_tiling_on_sc=True, needs_layout_passes=False),
)
def _sc_fused(idx_hbm, tablet_hbm, gamma_hbm, beta_hbm, out_hbm,
              tbl_t, g_v, b_v, idx_v, rows_v, sem, gsem, osem):
    sid = lax.axis_index("s")
    wid = sid * _NC + lax.axis_index("c")
    base = wid * _B_PER_W

    with jax.named_scope("stage"):
        cp_idx = pltpu.async_copy(idx_hbm.at[pl.ds(base, _B_PER_W)], idx_v, sem)
        cp_tbl = pltpu.async_copy(tablet_hbm, tbl_t, sem)
        cp_g = pltpu.async_copy(gamma_hbm, g_v, gsem)
        cp_b = pltpu.async_copy(beta_hbm, b_v, gsem)
        cp_tbl.wait()

    ln_scope = jax.named_scope("normalize")
    ln_scope.__enter__()
    half = jnp.float32(0.5)
    threehalf = jnp.float32(1.5)
    inv_d = jnp.float32(1.0 / _EMBED_DIM)
    zeros = jnp.zeros((_LANES,), jnp.float32)

    @plsc.parallel_loop(0, _EMBED_DIM, carry=tuple([zeros] * (2 * _NG)))
    def stats(d, acc):
        out = []
        for gi in range(_NG):
            v = tbl_t[d, pl.ds(_group_off(gi), _LANES)]
            out.append(acc[gi] + v)
            out.append(acc[_NG + gi] + v * v)
        return tuple(out[0::2] + out[1::2])

    means = [stats[gi] * inv_d for gi in range(_NG)]
    rstds = []
    for gi in range(_NG):
        var = stats[_NG + gi] * inv_d - means[gi] * means[gi]
        v = var + jnp.float32(_EPS)
        i = plsc.bitcast(v, jnp.int32)
        i = jnp.int32(0x5F3759DF) - (i >> 1)
        y = plsc.bitcast(i, jnp.float32)
        for _ in range(3):
            y = y * (threehalf - half * v * y * y)
        rstds.append(y)

    tail_idx = lax.iota(jnp.int32, _LANES)
    tail_mask = tail_idx >= jnp.int32(_LANES - (_NUM_HW - _NFULL * _LANES))

    cp_g.wait()
    cp_b.wait()

    @plsc.parallel_loop(0, _EMBED_DIM)
    def _(d):
        d16 = jnp.full((_LANES,), d, jnp.int32)
        gd = plsc.load_gather(g_v, [d16])
        bd = plsc.load_gather(b_v, [d16])
        for gi in range(_NFULL):
            x = tbl_t[d, pl.ds(_LANES * gi, _LANES)]
            tbl_t[d, pl.ds(_LANES * gi, _LANES)] = (
                (x - means[gi]) * rstds[gi] * gd + bd)
        x = tbl_t[d, pl.ds(_OVER, _LANES)]
        y = (x - means[_NFULL]) * rstds[_NFULL] * gd + bd
        plsc.store_scatter(tbl_t, [d16, tail_idx + _OVER], y, mask=tail_mask)

    ln_scope.__exit__(None, None, None)

    cp_idx.wait()

    copies = []
    for c in range(_NCHUNK):
        with jax.named_scope("expand"):
            @plsc.parallel_loop(c * _G_CHUNK, (c + 1) * _G_CHUNK)
            def _(bg):
                bo = bg * _LANES
                idx16 = idx_v[pl.ds(bo, _LANES)]

                @plsc.parallel_loop(0, _EMBED_DIM)
                def _(d):
                    d16 = jnp.full((_LANES,), d, jnp.int32)
                    rows_v[d, pl.ds(bo, _LANES)] = plsc.load_gather(
                        tbl_t, [d16, idx16])

        with jax.named_scope("flush"):
            copies.append(pltpu.async_copy(
                rows_v.at[:, pl.ds(c * _B_CHUNK, _B_CHUNK)],
                out_hbm.at[:, pl.ds(base + c * _B_CHUNK, _B_CHUNK)],
                osem))
    with jax.named_scope("drain"):
        for cp in copies:
            cp.wait()


def kernel(hw_indices, table, gamma, beta):
    out_t = _sc_fused(hw_indices.astype(jnp.int32), table.T, gamma, beta)
    return out_t.T


# speedup vs baseline: 1.8639x; 1.1107x over previous
import functools

import jax
import jax.numpy as jnp
from jax import lax
from jax.experimental import pallas as pl
from jax.experimental.pallas import tpu as pltpu
from jax.experimental.pallas import tpu_sc as plsc

_EPS = 1e-5

_NUM_HW = 100
_EMBED_DIM = 64
_BATCH = 16384

_info = plsc.get_sparse_core_info()
_NC, _NS = _info.num_cores, _info.num_subcores
_NW = _NC * _NS
_B_PER_W = _BATCH // _NW
_LANES = 16
_NFULL = _NUM_HW // _LANES
_OVER = _NUM_HW - _LANES
_NG = _NFULL + 1
_NCHUNK = 4
_B_CHUNK = _B_PER_W // _NCHUNK
_G_CHUNK = _B_CHUNK // _LANES

_mesh = plsc.VectorSubcoreMesh(core_axis_name="c", subcore_axis_name="s")


def _group_off(gi):
    return _LANES * gi if gi < _NFULL else _OVER


@functools.partial(
    pl.kernel,
    mesh=_mesh,
    out_type=jax.ShapeDtypeStruct((_EMBED_DIM, _BATCH), jnp.float32),
    scratch_types=[
        pltpu.VMEM((_EMBED_DIM, _NUM_HW), jnp.float32),
        pltpu.VMEM((_EMBED_DIM,), jnp.float32),
        pltpu.VMEM((_EMBED_DIM,), jnp.float32),
        pltpu.VMEM((_B_PER_W,), jnp.int32),
        pltpu.VMEM((_EMBED_DIM, _B_PER_W), jnp.float32),
        pltpu.SemaphoreType.DMA,
        pltpu.SemaphoreType.DMA,
        pltpu.SemaphoreType.DMA,
    ],
    compiler_params=pltpu.CompilerParams(
        use_tc_tiling_on_sc=True, needs_layout_passes=False),
)
def _sc_fused(idx_hbm, tablet_hbm, gamma_hbm, beta_hbm, out_hbm,
              tbl_t, g_v, b_v, idx_v, rows_v, sem, gsem, osem):
    sid = lax.axis_index("s")
    wid = sid * _NC + lax.axis_index("c")
    base = wid * _B_PER_W

    with jax.named_scope("stage"):
        cp_idx = pltpu.async_copy(idx_hbm.at[pl.ds(base, _B_PER_W)], idx_v, sem)
        cp_tbl = pltpu.async_copy(tablet_hbm, tbl_t, sem)
        cp_g = pltpu.async_copy(gamma_hbm, g_v, gsem)
        cp_b = pltpu.async_copy(beta_hbm, b_v, gsem)
        cp_tbl.wait()

    ln_scope = jax.named_scope("normalize")
    ln_scope.__enter__()
    half = jnp.float32(0.5)
    threehalf = jnp.float32(1.5)
    inv_d = jnp.float32(1.0 / _EMBED_DIM)
    zeros = jnp.zeros((_LANES,), jnp.float32)

    @plsc.parallel_loop(0, _EMBED_DIM, carry=tuple([zeros] * (2 * _NG)))
    def stats(d, acc):
        out = []
        for gi in range(_NG):
            v = tbl_t[d, pl.ds(_group_off(gi), _LANES)]
            out.append(acc[gi] + v)
            out.append(acc[_NG + gi] + v * v)
        return tuple(out[0::2] + out[1::2])

    means = [stats[gi] * inv_d for gi in range(_NG)]
    rstds = []
    for gi in range(_NG):
        var = stats[_NG + gi] * inv_d - means[gi] * means[gi]
        v = var + jnp.float32(_EPS)
        i = plsc.bitcast(v, jnp.int32)
        i = jnp.int32(0x5F3759DF) - (i >> 1)
        y = plsc.bitcast(i, jnp.float32)
        for _ in range(3):
            y = y * (threehalf - half * v * y * y)
        rstds.append(y)

    tail_idx = lax.iota(jnp.int32, _LANES)
    tail_mask = tail_idx >= jnp.int32(_LANES - (_NUM_HW - _NFULL * _LANES))

    cp_g.wait()
    cp_b.wait()

    @plsc.parallel_loop(0, _EMBED_DIM)
    def _(d):
        d16 = jnp.full((_LANES,), d, jnp.int32)
        gd = plsc.load_gather(g_v, [d16])
        bd = plsc.load_gather(b_v, [d16])
        for gi in range(_NFULL):
            x = tbl_t[d, pl.ds(_LANES * gi, _LANES)]
            tbl_t[d, pl.ds(_LANES * gi, _LANES)] = (
                (x - means[gi]) * rstds[gi] * gd + bd)
        x = tbl_t[d, pl.ds(_OVER, _LANES)]
        y = (x - means[_NFULL]) * rstds[_NFULL] * gd + bd
        plsc.store_scatter(tbl_t, [d16, tail_idx + _OVER], y, mask=tail_mask)

    ln_scope.__exit__(None, None, None)

    cp_idx.wait()

    copies = []
    for c in range(_NCHUNK):
        with jax.named_scope("expand"):
            @plsc.parallel_loop(c * _G_CHUNK, (c + 1) * _G_CHUNK)
            def _(bg):
                bo = bg * _LANES
                idx16 = idx_v[pl.ds(bo, _LANES)]

                @plsc.parallel_loop(0, _EMBED_DIM, unroll=16)
                def _(d):
                    d16 = jnp.full((_LANES,), d, jnp.int32)
                    rows_v[d, pl.ds(bo, _LANES)] = plsc.load_gather(
                        tbl_t, [d16, idx16])

        with jax.named_scope("flush"):
            copies.append(pltpu.async_copy(
                rows_v.at[:, pl.ds(c * _B_CHUNK, _B_CHUNK)],
                out_hbm.at[:, pl.ds(base + c * _B_CHUNK, _B_CHUNK)],
                osem))
    with jax.named_scope("drain"):
        for cp in copies:
            cp.wait()


def kernel(hw_indices, table, gamma, beta):
    out_t = _sc_fused(hw_indices.astype(jnp.int32), table.T, gamma, beta)
    return out_t.T


# speedup vs baseline: 1.9218x; 1.0311x over previous
import functools

import jax
import jax.numpy as jnp
from jax import lax
from jax.experimental import pallas as pl
from jax.experimental.pallas import tpu as pltpu
from jax.experimental.pallas import tpu_sc as plsc

_EPS = 1e-5

_NUM_HW = 100
_EMBED_DIM = 64
_BATCH = 16384

_info = plsc.get_sparse_core_info()
_NC, _NS = _info.num_cores, _info.num_subcores
_NW = _NC * _NS
_B_PER_W = _BATCH // _NW
_LANES = 16
_NFULL = _NUM_HW // _LANES
_OVER = _NUM_HW - _LANES
_NG = _NFULL + 1
_NCHUNK = 4
_B_CHUNK = _B_PER_W // _NCHUNK
_G_CHUNK = _B_CHUNK // _LANES

_mesh = plsc.VectorSubcoreMesh(core_axis_name="c", subcore_axis_name="s")


def _group_off(gi):
    return _LANES * gi if gi < _NFULL else _OVER


@functools.partial(
    pl.kernel,
    mesh=_mesh,
    out_type=jax.ShapeDtypeStruct((_EMBED_DIM, _BATCH), jnp.float32),
    scratch_types=[
        pltpu.VMEM((_EMBED_DIM, _NUM_HW), jnp.float32),
        pltpu.VMEM((_EMBED_DIM,), jnp.float32),
        pltpu.VMEM((_EMBED_DIM,), jnp.float32),
        pltpu.VMEM((_B_PER_W,), jnp.int32),
        pltpu.VMEM((_EMBED_DIM, _B_PER_W), jnp.float32),
        pltpu.SemaphoreType.DMA,
        pltpu.SemaphoreType.DMA,
        pltpu.SemaphoreType.DMA,
    ],
    compiler_params=pltpu.CompilerParams(
        use_tc_tiling_on_sc=True, needs_layout_passes=False),
)
def _sc_fused(idx_hbm, tablet_hbm, gamma_hbm, beta_hbm, out_hbm,
              tbl_t, g_v, b_v, idx_v, rows_v, sem, gsem, osem):
    sid = lax.axis_index("s")
    wid = sid * _NC + lax.axis_index("c")
    base = wid * _B_PER_W

    with jax.named_scope("stage"):
        cp_idx = pltpu.async_copy(idx_hbm.at[pl.ds(base, _B_PER_W)], idx_v, sem)
        cp_g = pltpu.async_copy(gamma_hbm, g_v, gsem)
        cp_b = pltpu.async_copy(beta_hbm, b_v, gsem)
        tbl_copies = []
        for j in range(_EMBED_DIM // 8):
            blk = ((sid + j) % (_EMBED_DIM // 8)) * 8
            tbl_copies.append(pltpu.async_copy(
                tablet_hbm.at[pl.ds(blk, 8)], tbl_t.at[pl.ds(blk, 8)], sem))
        for cp in tbl_copies:
            cp.wait()

    ln_scope = jax.named_scope("normalize")
    ln_scope.__enter__()
    half = jnp.float32(0.5)
    threehalf = jnp.float32(1.5)
    inv_d = jnp.float32(1.0 / _EMBED_DIM)
    zeros = jnp.zeros((_LANES,), jnp.float32)

    @plsc.parallel_loop(0, _EMBED_DIM, carry=tuple([zeros] * (2 * _NG)))
    def stats(d, acc):
        out = []
        for gi in range(_NG):
            v = tbl_t[d, pl.ds(_group_off(gi), _LANES)]
            out.append(acc[gi] + v)
            out.append(acc[_NG + gi] + v * v)
        return tuple(out[0::2] + out[1::2])

    means = [stats[gi] * inv_d for gi in range(_NG)]
    rstds = []
    for gi in range(_NG):
        var = stats[_NG + gi] * inv_d - means[gi] * means[gi]
        v = var + jnp.float32(_EPS)
        i = plsc.bitcast(v, jnp.int32)
        i = jnp.int32(0x5F3759DF) - (i >> 1)
        y = plsc.bitcast(i, jnp.float32)
        for _ in range(3):
            y = y * (threehalf - half * v * y * y)
        rstds.append(y)

    tail_idx = lax.iota(jnp.int32, _LANES)
    tail_mask = tail_idx >= jnp.int32(_LANES - (_NUM_HW - _NFULL * _LANES))

    cp_g.wait()
    cp_b.wait()

    @plsc.parallel_loop(0, _EMBED_DIM)
    def _(d):
        d16 = jnp.full((_LANES,), d, jnp.int32)
        gd = plsc.load_gather(g_v, [d16])
        bd = plsc.load_gather(b_v, [d16])
        for gi in range(_NFULL):
            x = tbl_t[d, pl.ds(_LANES * gi, _LANES)]
            tbl_t[d, pl.ds(_LANES * gi, _LANES)] = (
                (x - means[gi]) * rstds[gi] * gd + bd)
        x = tbl_t[d, pl.ds(_OVER, _LANES)]
        y = (x - means[_NFULL]) * rstds[_NFULL] * gd + bd
        plsc.store_scatter(tbl_t, [d16, tail_idx + _OVER], y, mask=tail_mask)

    ln_scope.__exit__(None, None, None)

    cp_idx.wait()

    copies = []
    for c in range(_NCHUNK):
        with jax.named_scope("expand"):
            @plsc.parallel_loop(c * _G_CHUNK, (c + 1) * _G_CHUNK)
            def _(bg):
                bo = bg * _LANES
                idx16 = idx_v[pl.ds(bo, _LANES)]

                @plsc.parallel_loop(0, _EMBED_DIM, unroll=16)
                def _(d):
                    rows_v[d, pl.ds(bo, _LANES)] = plsc.load_gather(
                        tbl_t.at[d], [idx16])

        with jax.named_scope("flush"):
            copies.append(pltpu.async_copy(
                rows_v.at[:, pl.ds(c * _B_CHUNK, _B_CHUNK)],
                out_hbm.at[:, pl.ds(base + c * _B_CHUNK, _B_CHUNK)],
                osem))
    with jax.named_scope("drain"):
        for cp in copies:
            cp.wait()


def kernel(hw_indices, table, gamma, beta):
    out_t = _sc_fused(hw_indices.astype(jnp.int32), table.T, gamma, beta)
    return out_t.T


# speedup vs baseline: 1.9241x; 1.0012x over previous
import functools

import jax
import jax.numpy as jnp
from jax import lax
from jax.experimental import pallas as pl
from jax.experimental.pallas import tpu as pltpu
from jax.experimental.pallas import tpu_sc as plsc

_EPS = 1e-5

_NUM_HW = 100
_EMBED_DIM = 64
_BATCH = 16384

_info = plsc.get_sparse_core_info()
_NC, _NS = _info.num_cores, _info.num_subcores
_NW = _NC * _NS
_B_PER_W = _BATCH // _NW
_LANES = 16
_NFULL = _NUM_HW // _LANES
_OVER = _NUM_HW - _LANES
_NG = _NFULL + 1
_NCHUNK = 4
_B_CHUNK = _B_PER_W // _NCHUNK
_G_CHUNK = _B_CHUNK // _LANES

_mesh = plsc.VectorSubcoreMesh(core_axis_name="c", subcore_axis_name="s")


def _group_off(gi):
    return _LANES * gi if gi < _NFULL else _OVER


@functools.partial(
    pl.kernel,
    mesh=_mesh,
    out_type=jax.ShapeDtypeStruct((_EMBED_DIM, _BATCH), jnp.float32),
    scratch_types=[
        pltpu.VMEM((_EMBED_DIM, _NUM_HW), jnp.float32),
        pltpu.VMEM((_EMBED_DIM,), jnp.float32),
        pltpu.VMEM((_EMBED_DIM,), jnp.float32),
        pltpu.VMEM((_B_PER_W,), jnp.int32),
        pltpu.VMEM((_EMBED_DIM, _B_PER_W), jnp.float32),
        pltpu.SemaphoreType.DMA,
        pltpu.SemaphoreType.DMA,
        pltpu.SemaphoreType.DMA,
    ],
    compiler_params=pltpu.CompilerParams(
        use_tc_tiling_on_sc=True, needs_layout_passes=False),
)
def _sc_fused(idx_hbm, tablet_hbm, gamma_hbm, beta_hbm, out_hbm,
              tbl_t, g_v, b_v, idx_v, rows_v, sem, gsem, osem):
    sid = lax.axis_index("s")
    wid = sid * _NC + lax.axis_index("c")
    base = wid * _B_PER_W

    with jax.named_scope("stage"):
        cp_idx = pltpu.async_copy(idx_hbm.at[pl.ds(base, _B_PER_W)], idx_v, sem)
        cp_g = pltpu.async_copy(gamma_hbm, g_v, gsem)
        cp_b = pltpu.async_copy(beta_hbm, b_v, gsem)
        cp_tbl = pltpu.async_copy(tablet_hbm, tbl_t, sem)
        cp_tbl.wait()

    ln_scope = jax.named_scope("normalize")
    ln_scope.__enter__()
    half = jnp.float32(0.5)
    threehalf = jnp.float32(1.5)
    inv_d = jnp.float32(1.0 / _EMBED_DIM)
    zeros = jnp.zeros((_LANES,), jnp.float32)

    @plsc.parallel_loop(0, _EMBED_DIM, carry=tuple([zeros] * (2 * _NG)))
    def stats(d, acc):
        out = []
        for gi in range(_NG):
            v = tbl_t[d, pl.ds(_group_off(gi), _LANES)]
            out.append(acc[gi] + v)
            out.append(acc[_NG + gi] + v * v)
        return tuple(out[0::2] + out[1::2])

    means = [stats[gi] * inv_d for gi in range(_NG)]
    rstds = []
    for gi in range(_NG):
        var = stats[_NG + gi] * inv_d - means[gi] * means[gi]
        v = var + jnp.float32(_EPS)
        i = plsc.bitcast(v, jnp.int32)
        i = jnp.int32(0x5F3759DF) - (i >> 1)
        y = plsc.bitcast(i, jnp.float32)
        for _ in range(3):
            y = y * (threehalf - half * v * y * y)
        rstds.append(y)

    tail_idx = lax.iota(jnp.int32, _LANES)
    tail_mask = tail_idx >= jnp.int32(_LANES - (_NUM_HW - _NFULL * _LANES))

    cp_g.wait()
    cp_b.wait()

    @plsc.parallel_loop(0, _EMBED_DIM)
    def _(d):
        d16 = jnp.full((_LANES,), d, jnp.int32)
        gd = plsc.load_gather(g_v, [d16])
        bd = plsc.load_gather(b_v, [d16])
        for gi in range(_NFULL):
            x = tbl_t[d, pl.ds(_LANES * gi, _LANES)]
            tbl_t[d, pl.ds(_LANES * gi, _LANES)] = (
                (x - means[gi]) * rstds[gi] * gd + bd)
        x = tbl_t[d, pl.ds(_OVER, _LANES)]
        y = (x - means[_NFULL]) * rstds[_NFULL] * gd + bd
        plsc.store_scatter(tbl_t, [d16, tail_idx + _OVER], y, mask=tail_mask)

    ln_scope.__exit__(None, None, None)

    cp_idx.wait()

    copies = []
    for c in range(_NCHUNK):
        with jax.named_scope("expand"):
            @plsc.parallel_loop(c * _G_CHUNK, (c + 1) * _G_CHUNK)
            def _(bg):
                bo = bg * _LANES
                idx16 = idx_v[pl.ds(bo, _LANES)]

                @plsc.parallel_loop(0, _EMBED_DIM, unroll=16)
                def _(d):
                    rows_v[d, pl.ds(bo, _LANES)] = plsc.load_gather(
                        tbl_t.at[d], [idx16])

        with jax.named_scope("flush"):
            copies.append(pltpu.async_copy(
                rows_v.at[:, pl.ds(c * _B_CHUNK, _B_CHUNK)],
                out_hbm.at[:, pl.ds(base + c * _B_CHUNK, _B_CHUNK)],
                osem))
    with jax.named_scope("drain"):
        for cp in copies:
            cp.wait()


def kernel(hw_indices, table, gamma, beta):
    out_t = _sc_fused(hw_indices.astype(jnp.int32), table.T, gamma, beta)
    return out_t.T
